# baseline (speedup 1.0000x reference)
import functools
import jax
import jax.numpy as jnp
from jax import lax
from jax.experimental import pallas as pl
from jax.experimental.pallas import tpu as pltpu
from jax.experimental.pallas import tpu_sc as plsc

NPAD = 10240
EPAD = 327680
D = 128
NGR = 8
ROWS_PER_TILE = NPAD // 32
ACC_ROWS_PER_TILE = NPAD // 16
ECHUNK = 32
NCH = EPAD // 32 // ECHUNK
GSZ = NCH // 8
NSTREAM = 8


def _seg_body(g_hbm, src_hbm, dst_hbm, zeros_hbm, out_hbm,
              src_v, dst_v, r0, r1, r2, r3, r4, r5, r6, r7, acc,
              s0, s1, s2, s3, s4, s5, s6, s7):
    c = lax.axis_index("c")
    sid = lax.axis_index("s")
    wid = c * 16 + sid
    bufs = (r0, r1, r2, r3, r4, r5, r6, r7)
    sems = (s0, s1, s2, s3, s4, s5, s6, s7)
    pltpu.sync_copy(zeros_hbm, acc.at[pl.ds(sid * ACC_ROWS_PER_TILE, ACC_ROWS_PER_TILE)])
    plsc.subcore_barrier()

    for gi in range(NCH // GSZ):
        base = wid * NCH + gi * GSZ
        pltpu.sync_copy(src_hbm.at[pl.ds(base, GSZ)], src_v)
        pltpu.sync_copy(dst_hbm.at[pl.ds(base, GSZ)], dst_v)
        for u in range(NSTREAM):
            pltpu.make_async_copy(g_hbm.at[src_v.at[u]], bufs[u], sems[u]).start()

        def ring(i, c2):
            j = i * NSTREAM
            for u in range(NSTREAM):
                pltpu.make_async_copy(g_hbm.at[src_v.at[j + u]],
                                      bufs[u], sems[u]).wait()
                pltpu.sync_copy(bufs[u], acc.at[dst_v.at[j + u]], add=True)

                @pl.when(j + NSTREAM + u < GSZ)
                def _(u=u, j=j):
                    pltpu.make_async_copy(g_hbm.at[src_v.at[j + NSTREAM + u]],
                                          bufs[u], sems[u]).start()
            return c2

        lax.fori_loop(0, GSZ // NSTREAM, ring, 0, unroll=False)
    plsc.subcore_barrier()
    pltpu.sync_copy(
        acc.at[pl.ds(sid * ACC_ROWS_PER_TILE, ACC_ROWS_PER_TILE)],
        out_hbm.at[pl.ds(c * NPAD + sid * ACC_ROWS_PER_TILE, ACC_ROWS_PER_TILE)])


@functools.lru_cache(maxsize=1)
def _seg_parts_kernel():
    return pl.kernel(
        _seg_body,
        mesh=plsc.VectorSubcoreMesh(core_axis_name="c", subcore_axis_name="s"),
        out_type=jax.ShapeDtypeStruct((2 * NPAD, D), jnp.float32),
        scratch_types=[
            pltpu.VMEM((GSZ, ECHUNK), jnp.int32),
            pltpu.VMEM((GSZ, ECHUNK), jnp.int32),
            *([pltpu.VMEM((ECHUNK, D), jnp.float32)] * NSTREAM),
            pltpu.VMEM_SHARED((NPAD, D), jnp.float32),
            *([pltpu.SemaphoreType.DMA] * NSTREAM),
        ],
    )


def _seg_parts(g, src, dst, zeros640):
    return _seg_parts_kernel()(g, src, dst, zeros640)


_BLK = 1024
_NBLK = NPAD // _BLK


def _dense_body(p0, p1, g, Wn, Wr, b, p, valid, h_out, s_out):
    agg = p0[...] + p1[...]
    h = jnp.dot(agg, Wn[...], preferred_element_type=jnp.float32)
    h = h + jnp.dot(g[...], Wr[...], preferred_element_type=jnp.float32)
    h = jnp.maximum(h + b[...], 0.0)
    pv = p[...]
    nrm = jnp.sqrt(jnp.sum(pv * pv)) + 1e-12
    s = jnp.dot(h, pv, preferred_element_type=jnp.float32) / nrm
    s = jnp.where(valid[...] > 0, s, -jnp.inf)
    h_out[...] = h
    s_out[...] = s


def _dense(p0, p1, g, Wn, Wr, b, p, valid):
    return pl.pallas_call(
        _dense_body,
        grid=(_NBLK,),
        in_specs=[
            pl.BlockSpec((_BLK, D), lambda i: (i, 0)),
            pl.BlockSpec((_BLK, D), lambda i: (i, 0)),
            pl.BlockSpec((_BLK, D), lambda i: (i, 0)),
            pl.BlockSpec((D, D), lambda i: (0, 0)),
            pl.BlockSpec((D, D), lambda i: (0, 0)),
            pl.BlockSpec((1, D), lambda i: (0, 0)),
            pl.BlockSpec((D, 1), lambda i: (0, 0)),
            pl.BlockSpec((_BLK, 1), lambda i: (i, 0)),
        ],
        out_specs=[
            pl.BlockSpec((_BLK, D), lambda i: (i, 0)),
            pl.BlockSpec((_BLK, 1), lambda i: (i, 0)),
        ],
        out_shape=[
            jax.ShapeDtypeStruct((NPAD, D), jnp.float32),
            jax.ShapeDtypeStruct((NPAD, 1), jnp.float32),
        ],
    )(p0, p1, g, Wn, Wr, b, p, valid)


def _mask_body(s_ref, m_ref, *, k):
    s = s_ref[...]
    bits = lax.bitcast_convert_type(s, jnp.uint32)
    flip = jnp.where(bits >= jnp.uint32(0x80000000),
                     jnp.uint32(0xFFFFFFFF), jnp.uint32(0x80000000))
    u = bits ^ flip

    def bit_body(i, t):
        tt = t | (jnp.uint32(1) << (jnp.uint32(31) - i.astype(jnp.uint32)))
        cnt = jnp.sum((u >= tt).astype(jnp.int32))
        return jnp.where(cnt >= k, tt, t)

    t = lax.fori_loop(0, 32, bit_body, jnp.uint32(0))
    m_ref[...] = (u >= t).astype(jnp.float32)


def _mask(s80, k):
    return pl.pallas_call(
        functools.partial(_mask_body, k=k),
        out_shape=jax.ShapeDtypeStruct((NPAD // 128, 128), jnp.float32),
    )(s80)


def _read_body(h, s, m, bt, gp_out, gmp_out, gap_out, mx, sm, cnt):
    j = pl.program_id(0)

    @pl.when(j == 0)
    def _():
        mx[...] = jnp.full((NGR, D), -jnp.inf, jnp.float32)
        sm[...] = jnp.zeros((NGR, D), jnp.float32)
        cnt[...] = jnp.zeros((NGR, D), jnp.float32)

    hv = h[...]
    sv = s[...]
    mv = m[...]
    btv = bt[...]
    g = jnp.where(mv > 0, hv * jnp.tanh(sv), 0.0)
    gp_out[...] = g

    gids = lax.broadcasted_iota(jnp.int32, (1, NGR), 1).astype(jnp.float32)
    oh = ((btv == gids) & (mv > 0)).astype(jnp.float32)
    sm[...] += lax.dot_general(oh, g, (((0,), (0,)), ((), ())),
                               preferred_element_type=jnp.float32)
    cnt[...] += jnp.sum(oh, axis=0, keepdims=True).reshape(NGR, 1)

    rows = []
    for gi in range(NGR):
        rmask = (btv == jnp.float32(gi)) & (mv > 0)
        rows.append(jnp.max(jnp.where(rmask, g, -jnp.inf), axis=0, keepdims=True))
    mx[...] = jnp.maximum(mx[...], jnp.concatenate(rows, axis=0))

    @pl.when(j == pl.num_programs(0) - 1)
    def _():
        mxv = mx[...]
        gmp_out[...] = jnp.where(mxv == -jnp.inf, 0.0, mxv)
        gap_out[...] = sm[...] / jnp.maximum(cnt[...], 1.0)


def _readout(h, s, m, bt):
    return pl.pallas_call(
        _read_body,
        grid=(_NBLK,),
        in_specs=[
            pl.BlockSpec((_BLK, D), lambda i: (i, 0)),
            pl.BlockSpec((_BLK, 1), lambda i: (i, 0)),
            pl.BlockSpec((_BLK, 1), lambda i: (i, 0)),
            pl.BlockSpec((_BLK, 1), lambda i: (i, 0)),
        ],
        out_specs=[
            pl.BlockSpec((_BLK, D), lambda i: (i, 0)),
            pl.BlockSpec((NGR, D), lambda i: (0, 0)),
            pl.BlockSpec((NGR, D), lambda i: (0, 0)),
        ],
        out_shape=[
            jax.ShapeDtypeStruct((NPAD, D), jnp.float32),
            jax.ShapeDtypeStruct((NGR, D), jnp.float32),
            jax.ShapeDtypeStruct((NGR, D), jnp.float32),
        ],
        scratch_shapes=[
            pltpu.VMEM((NGR, D), jnp.float32),
            pltpu.VMEM((NGR, D), jnp.float32),
            pltpu.VMEM((NGR, D), jnp.float32),
        ],
    )(h, s, m, bt)


def _mlp_body(z, W1, b1, W2, b2, W3, b3, out):
    z1 = jnp.maximum(jnp.dot(z[...], W1[...], preferred_element_type=jnp.float32)
                     + b1[...], 0.0)
    z2 = jnp.maximum(jnp.dot(z1, W2[...], preferred_element_type=jnp.float32)
                     + b2[...], 0.0)
    z3 = jnp.dot(z2, W3[...], preferred_element_type=jnp.float32) + b3[...]
    mxv = jnp.max(z3, axis=1, keepdims=True)
    e = jnp.exp(z3 - mxv)
    out[...] = z3 - mxv - jnp.log(jnp.sum(e, axis=1, keepdims=True))


def _mlp(z, W1, b1, W2, b2, W3, b3):
    return pl.pallas_call(
        _mlp_body,
        out_shape=jax.ShapeDtypeStruct((NGR, W3.shape[1]), jnp.float32),
    )(z, W1, b1, W2, b2, W3, b3)


def kernel(x, edge_index, batch, Wr1, Wn1, b1, Wr2, Wn2, b2, Wr3, Wn3, b3,
           p1, p2, p3, Wl1, bl1, Wl2, bl2, Wl3, bl3):
    n = x.shape[0]
    e = edge_index.shape[1]

    g = jnp.pad(x, ((0, NPAD - n), (0, 0)))
    src = jnp.pad(edge_index[0], (0, EPAD - e),
                  constant_values=NPAD - 1).reshape(EPAD // ECHUNK, ECHUNK)
    dst = jnp.pad(edge_index[1], (0, EPAD - e),
                  constant_values=NPAD - 1).reshape(EPAD // ECHUNK, ECHUNK)
    zeros640 = jnp.zeros((ACC_ROWS_PER_TILE, D), jnp.float32)
    valid = jnp.pad(jnp.ones((n, 1), jnp.float32), ((0, NPAD - n), (0, 0)))
    btf = jnp.pad(batch, (0, NPAD - n),
                  constant_values=NGR).astype(jnp.float32).reshape(NPAD, 1)

    layers = [(Wn1, Wr1, b1, p1, 5000), (Wn2, Wr2, b2, p2, 2500),
              (Wn3, Wr3, b3, p3, 1250)]
    outs = []
    for Wn, Wr, b, p, k in layers:
        parts = _seg_parts(g, src, dst, zeros640)
        h, s = _dense(parts[:NPAD], parts[NPAD:], g, Wn, Wr,
                      b.reshape(1, D), p.reshape(D, 1), valid)
        m = _mask(s.reshape(NPAD // 128, 128), k).reshape(NPAD, 1)
        g, gmp, gap = _readout(h, s, m, btf)
        valid = m
        outs += [gmp, gap]

    z = jnp.concatenate(outs, axis=1)
    return _mlp(z, Wl1, bl1.reshape(1, -1), Wl2, bl2.reshape(1, -1),
                Wl3, bl3.reshape(1, -1))


# speedup vs baseline: 1.3176x; 1.3176x over previous
import functools
import jax
import jax.numpy as jnp
from jax import lax
from jax.experimental import pallas as pl
from jax.experimental.pallas import tpu as pltpu
from jax.experimental.pallas import tpu_sc as plsc

NPAD = 10240
EPAD = 327680
D = 128
NGR = 8
ROWS_PER_TILE = NPAD // 32
ACC_ROWS_PER_TILE = NPAD // 16
ECHUNK = 64
NCH = EPAD // 32 // ECHUNK
GSZ = NCH // 4


def _seg_body(g_hbm, src_hbm, dst_hbm, zeros_hbm, out_hbm,
              src_v, dst_v, r0, r1, r2, r3, acc, s0, s1, s2, s3):
    c = lax.axis_index("c")
    sid = lax.axis_index("s")
    wid = c * 16 + sid
    bufs = (r0, r1, r2, r3)
    sems = (s0, s1, s2, s3)
    pltpu.sync_copy(zeros_hbm, acc.at[pl.ds(sid * ACC_ROWS_PER_TILE, ACC_ROWS_PER_TILE)])
    plsc.subcore_barrier()

    for gi in range(NCH // GSZ):
        base = wid * NCH + gi * GSZ
        pltpu.sync_copy(src_hbm.at[pl.ds(base, GSZ)], src_v)
        pltpu.sync_copy(dst_hbm.at[pl.ds(base, GSZ)], dst_v)
        for u in range(4):
            pltpu.make_async_copy(g_hbm.at[src_v.at[u]], bufs[u], sems[u]).start()

        def quad(i, c2):
            j = i * 4
            for u in range(4):
                pltpu.make_async_copy(g_hbm.at[src_v.at[j + u]],
                                      bufs[u], sems[u]).wait()
                pltpu.sync_copy(bufs[u], acc.at[dst_v.at[j + u]], add=True)

                @pl.when(j + 4 + u < GSZ)
                def _(u=u, j=j):
                    pltpu.make_async_copy(g_hbm.at[src_v.at[j + 4 + u]],
                                          bufs[u], sems[u]).start()
            return c2

        lax.fori_loop(0, GSZ // 4, quad, 0, unroll=False)
    plsc.subcore_barrier()
    pltpu.sync_copy(
        acc.at[pl.ds(sid * ACC_ROWS_PER_TILE, ACC_ROWS_PER_TILE)],
        out_hbm.at[pl.ds(c * NPAD + sid * ACC_ROWS_PER_TILE, ACC_ROWS_PER_TILE)])


@functools.lru_cache(maxsize=1)
def _seg_parts_kernel():
    return pl.kernel(
        _seg_body,
        mesh=plsc.VectorSubcoreMesh(core_axis_name="c", subcore_axis_name="s"),
        out_type=jax.ShapeDtypeStruct((2 * NPAD, D), jnp.float32),
        scratch_types=[
            pltpu.VMEM((GSZ, ECHUNK), jnp.int32),
            pltpu.VMEM((GSZ, ECHUNK), jnp.int32),
            pltpu.VMEM((ECHUNK, D), jnp.float32),
            pltpu.VMEM((ECHUNK, D), jnp.float32),
            pltpu.VMEM((ECHUNK, D), jnp.float32),
            pltpu.VMEM((ECHUNK, D), jnp.float32),
            pltpu.VMEM_SHARED((NPAD, D), jnp.float32),
            pltpu.SemaphoreType.DMA,
            pltpu.SemaphoreType.DMA,
            pltpu.SemaphoreType.DMA,
            pltpu.SemaphoreType.DMA,
        ],
    )


def _seg_parts(g, src, dst, zeros640):
    return _seg_parts_kernel()(g, src, dst, zeros640)


_BLK = 1024
_NBLK = NPAD // _BLK


def _dense_body(p0, p1, g, Wn, Wr, b, p, valid, h_out, s_out):
    agg = p0[...] + p1[...]
    h = jnp.dot(agg, Wn[...], preferred_element_type=jnp.float32)
    h = h + jnp.dot(g[...], Wr[...], preferred_element_type=jnp.float32)
    h = jnp.maximum(h + b[...], 0.0)
    pv = p[...]
    nrm = jnp.sqrt(jnp.sum(pv * pv)) + 1e-12
    s = jnp.dot(h, pv, preferred_element_type=jnp.float32) / nrm
    s = jnp.where(valid[...] > 0, s, -jnp.inf)
    h_out[...] = h
    s_out[...] = s


def _dense(p0, p1, g, Wn, Wr, b, p, valid):
    return pl.pallas_call(
        _dense_body,
        grid=(_NBLK,),
        in_specs=[
            pl.BlockSpec((_BLK, D), lambda i: (i, 0)),
            pl.BlockSpec((_BLK, D), lambda i: (i, 0)),
            pl.BlockSpec((_BLK, D), lambda i: (i, 0)),
            pl.BlockSpec((D, D), lambda i: (0, 0)),
            pl.BlockSpec((D, D), lambda i: (0, 0)),
            pl.BlockSpec((1, D), lambda i: (0, 0)),
            pl.BlockSpec((D, 1), lambda i: (0, 0)),
            pl.BlockSpec((_BLK, 1), lambda i: (i, 0)),
        ],
        out_specs=[
            pl.BlockSpec((_BLK, D), lambda i: (i, 0)),
            pl.BlockSpec((_BLK, 1), lambda i: (i, 0)),
        ],
        out_shape=[
            jax.ShapeDtypeStruct((NPAD, D), jnp.float32),
            jax.ShapeDtypeStruct((NPAD, 1), jnp.float32),
        ],
    )(p0, p1, g, Wn, Wr, b, p, valid)


def _mask_body(s_ref, m_ref, *, k):
    s = s_ref[...]
    bits = lax.bitcast_convert_type(s, jnp.uint32)
    flip = jnp.where(bits >= jnp.uint32(0x80000000),
                     jnp.uint32(0xFFFFFFFF), jnp.uint32(0x80000000))
    u = bits ^ flip

    def bit_body(i, t):
        tt = t | (jnp.uint32(1) << (jnp.uint32(31) - i.astype(jnp.uint32)))
        cnt = jnp.sum((u >= tt).astype(jnp.int32))
        return jnp.where(cnt >= k, tt, t)

    t = lax.fori_loop(0, 32, bit_body, jnp.uint32(0))
    m_ref[...] = (u >= t).astype(jnp.float32)


def _mask(s80, k):
    return pl.pallas_call(
        functools.partial(_mask_body, k=k),
        out_shape=jax.ShapeDtypeStruct((NPAD // 128, 128), jnp.float32),
    )(s80)


def _read_body(h, s, m, bt, gp_out, gmp_out, gap_out, mx, sm, cnt):
    j = pl.program_id(0)

    @pl.when(j == 0)
    def _():
        mx[...] = jnp.full((NGR, D), -jnp.inf, jnp.float32)
        sm[...] = jnp.zeros((NGR, D), jnp.float32)
        cnt[...] = jnp.zeros((NGR, D), jnp.float32)

    hv = h[...]
    sv = s[...]
    mv = m[...]
    btv = bt[...]
    g = jnp.where(mv > 0, hv * jnp.tanh(sv), 0.0)
    gp_out[...] = g

    gids = lax.broadcasted_iota(jnp.int32, (1, NGR), 1).astype(jnp.float32)
    oh = ((btv == gids) & (mv > 0)).astype(jnp.float32)
    sm[...] += lax.dot_general(oh, g, (((0,), (0,)), ((), ())),
                               preferred_element_type=jnp.float32)
    cnt[...] += jnp.sum(oh, axis=0, keepdims=True).reshape(NGR, 1)

    rows = []
    for gi in range(NGR):
        rmask = (btv == jnp.float32(gi)) & (mv > 0)
        rows.append(jnp.max(jnp.where(rmask, g, -jnp.inf), axis=0, keepdims=True))
    mx[...] = jnp.maximum(mx[...], jnp.concatenate(rows, axis=0))

    @pl.when(j == pl.num_programs(0) - 1)
    def _():
        mxv = mx[...]
        gmp_out[...] = jnp.where(mxv == -jnp.inf, 0.0, mxv)
        gap_out[...] = sm[...] / jnp.maximum(cnt[...], 1.0)


def _readout(h, s, m, bt):
    return pl.pallas_call(
        _read_body,
        grid=(_NBLK,),
        in_specs=[
            pl.BlockSpec((_BLK, D), lambda i: (i, 0)),
            pl.BlockSpec((_BLK, 1), lambda i: (i, 0)),
            pl.BlockSpec((_BLK, 1), lambda i: (i, 0)),
            pl.BlockSpec((_BLK, 1), lambda i: (i, 0)),
        ],
        out_specs=[
            pl.BlockSpec((_BLK, D), lambda i: (i, 0)),
            pl.BlockSpec((NGR, D), lambda i: (0, 0)),
            pl.BlockSpec((NGR, D), lambda i: (0, 0)),
        ],
        out_shape=[
            jax.ShapeDtypeStruct((NPAD, D), jnp.float32),
            jax.ShapeDtypeStruct((NGR, D), jnp.float32),
            jax.ShapeDtypeStruct((NGR, D), jnp.float32),
        ],
        scratch_shapes=[
            pltpu.VMEM((NGR, D), jnp.float32),
            pltpu.VMEM((NGR, D), jnp.float32),
            pltpu.VMEM((NGR, D), jnp.float32),
        ],
    )(h, s, m, bt)


def _mlp_body(z, W1, b1, W2, b2, W3, b3, out):
    z1 = jnp.maximum(jnp.dot(z[...], W1[...], preferred_element_type=jnp.float32)
                     + b1[...], 0.0)
    z2 = jnp.maximum(jnp.dot(z1, W2[...], preferred_element_type=jnp.float32)
                     + b2[...], 0.0)
    z3 = jnp.dot(z2, W3[...], preferred_element_type=jnp.float32) + b3[...]
    mxv = jnp.max(z3, axis=1, keepdims=True)
    e = jnp.exp(z3 - mxv)
    out[...] = z3 - mxv - jnp.log(jnp.sum(e, axis=1, keepdims=True))


def _mlp(z, W1, b1, W2, b2, W3, b3):
    return pl.pallas_call(
        _mlp_body,
        out_shape=jax.ShapeDtypeStruct((NGR, W3.shape[1]), jnp.float32),
    )(z, W1, b1, W2, b2, W3, b3)


def kernel(x, edge_index, batch, Wr1, Wn1, b1, Wr2, Wn2, b2, Wr3, Wn3, b3,
           p1, p2, p3, Wl1, bl1, Wl2, bl2, Wl3, bl3):
    n = x.shape[0]
    e = edge_index.shape[1]

    g = jnp.pad(x, ((0, NPAD - n), (0, 0)))
    src = jnp.pad(edge_index[0], (0, EPAD - e),
                  constant_values=NPAD - 1).reshape(EPAD // ECHUNK, ECHUNK)
    dst = jnp.pad(edge_index[1], (0, EPAD - e),
                  constant_values=NPAD - 1).reshape(EPAD // ECHUNK, ECHUNK)
    zeros640 = jnp.zeros((ACC_ROWS_PER_TILE, D), jnp.float32)
    valid = jnp.pad(jnp.ones((n, 1), jnp.float32), ((0, NPAD - n), (0, 0)))
    btf = jnp.pad(batch, (0, NPAD - n),
                  constant_values=NGR).astype(jnp.float32).reshape(NPAD, 1)

    layers = [(Wn1, Wr1, b1, p1, 5000), (Wn2, Wr2, b2, p2, 2500),
              (Wn3, Wr3, b3, p3, 1250)]
    outs = []
    for Wn, Wr, b, p, k in layers:
        parts = _seg_parts(g, src, dst, zeros640)
        h, s = _dense(parts[:NPAD], parts[NPAD:], g, Wn, Wr,
                      b.reshape(1, D), p.reshape(D, 1), valid)
        m = _mask(s.reshape(NPAD // 128, 128), k).reshape(NPAD, 1)
        g, gmp, gap = _readout(h, s, m, btf)
        valid = m
        outs += [gmp, gap]

    z = jnp.concatenate(outs, axis=1)
    return _mlp(z, Wl1, bl1.reshape(1, -1), Wl2, bl2.reshape(1, -1),
                Wl3, bl3.reshape(1, -1))
